# Initial kernel scaffold; baseline (speedup 1.0000x reference)
import jax
import jax.numpy as jnp
from jax.experimental import pallas as pl


def kernel(x, w_enc, w_dec):
    raise NotImplementedError("write your pallas kernel here")


# speedup vs baseline: 1.1531x; 1.1531x over previous
import functools

import numpy as np
import jax
import jax.numpy as jnp
from jax.experimental import pallas as pl
from jax.experimental.pallas import tpu as pltpu


def _se_fused_kernel(x_ref, w_enc_ref, w_dec_ref, o_ref, *, inv_s):
    x = x_ref[0].astype(jnp.float32)
    mean = jnp.sum(x, axis=1, keepdims=True) * inv_s
    z = jnp.dot(w_enc_ref[...], mean, preferred_element_type=jnp.float32)
    z = jnp.maximum(z, 0.0)
    a = jnp.dot(w_dec_ref[...], z, preferred_element_type=jnp.float32)
    a = 1.0 / (1.0 + jnp.exp(-a))
    o_ref[0] = (x * a).astype(o_ref.dtype)


def kernel(x, w_enc, w_dec):
    orig_shape = x.shape
    N, C = int(x.shape[0]), int(x.shape[1])
    S = int(np.prod(x.shape[2:]))
    Cr = int(w_enc.shape[0])

    x3 = x.reshape(N, C, S)

    y3 = pl.pallas_call(
        functools.partial(_se_fused_kernel, inv_s=1.0 / float(S)),
        out_shape=jax.ShapeDtypeStruct((N, C, S), x.dtype),
        grid=(N,),
        in_specs=[
            pl.BlockSpec((1, C, S), lambda n: (n, 0, 0)),
            pl.BlockSpec((Cr, C), lambda n: (0, 0)),
            pl.BlockSpec((C, Cr), lambda n: (0, 0)),
        ],
        out_specs=pl.BlockSpec((1, C, S), lambda n: (n, 0, 0)),
        compiler_params=pltpu.CompilerParams(
            dimension_semantics=("parallel",),
            vmem_limit_bytes=48 * 1024 * 1024,
        ),
    )(x3, w_enc, w_dec)

    return y3.reshape(orig_shape)
